# baseline (speedup 1.0000x reference)
import functools

import jax
import jax.numpy as jnp
from jax import lax
from jax.experimental import pallas as pl
from jax.experimental.pallas import tpu as pltpu
from jax.experimental.pallas import tpu_sc as plsc

NUM_E = 1024
DIM = 64
N_TOK = 16 * 24 * 24
TILE = 3072
GRID = N_TOK // TILE

NC = 2
NS = 16
NW = NC * NS
B_PER_W = N_TOK // NW
N_CHUNK = 3
CHUNK = B_PER_W // N_CHUNK


def _argmin_body(x_ref, emb_t_ref, idx_ref):
    x = x_ref[...]
    et = emb_t_ref[...]
    scores2 = lax.dot_general(
        x, et + et, (((1,), (0,)), ((), ())), preferred_element_type=jnp.float32
    )
    x2 = jnp.sum(x * x, axis=1, keepdims=True)
    e2 = jnp.sum(et * et, axis=0, keepdims=True)
    best_v = x2 + e2[:, 0:128] - scores2[:, 0:128]
    best_g = jnp.zeros((TILE, 128), jnp.float32)
    for g in range(1, NUM_E // 128):
        v = x2 + e2[:, 128 * g:128 * (g + 1)] - scores2[:, 128 * g:128 * (g + 1)]
        upd = v < best_v
        best_v = jnp.where(upd, v, best_v)
        best_g = jnp.where(upd, jnp.float32(g), best_g)
    lane = lax.broadcasted_iota(jnp.int32, (TILE, 128), 1).astype(jnp.float32)
    best_c = best_g * 128.0 + lane
    bv_t = best_v.T
    bc_t = best_c.T
    m_t = jnp.min(bv_t, axis=0, keepdims=True)
    idx_f = jnp.min(
        jnp.where(bv_t == m_t, bc_t, jnp.float32(NUM_E)), axis=0
    )
    idx_ref[...] = idx_f.astype(jnp.int32)


_argmin_call = pl.pallas_call(
    _argmin_body,
    grid=(GRID,),
    in_specs=[
        pl.BlockSpec((TILE, DIM), lambda i: (i, 0)),
        pl.BlockSpec((DIM, NUM_E), lambda i: (0, 0)),
    ],
    out_specs=pl.BlockSpec((TILE,), lambda i: (i,)),
    out_shape=jax.ShapeDtypeStruct((N_TOK,), jnp.int32),
)


PAD_D = 128


def _make_gather(n_tok):
    b_per_w = n_tok // NW

    @functools.partial(
        pl.kernel,
        mesh=plsc.VectorSubcoreMesh(core_axis_name="c", subcore_axis_name="s"),
        out_type=jax.ShapeDtypeStruct((n_tok, PAD_D), jnp.float32),
        scratch_types=[
            pltpu.VMEM((b_per_w,), jnp.int32),
            pltpu.VMEM((b_per_w, PAD_D), jnp.float32),
            pltpu.SemaphoreType.DMA,
            pltpu.SemaphoreType.DMA,
            pltpu.SemaphoreType.DMA,
        ],
    )
    def _sc_gather(table_hbm, idx_hbm, out_hbm, idx_v, rows_v, sem, sem1, semw):
        wid = lax.axis_index("s") * NC + lax.axis_index("c")
        base = wid * b_per_w
        half = b_per_w // 2
        pltpu.sync_copy(idx_hbm.at[pl.ds(base, b_per_w)], idx_v)
        g0 = pltpu.async_copy(
            table_hbm.at[idx_v.at[pl.ds(0, half)]],
            rows_v.at[pl.ds(0, half)], sem)
        g1 = pltpu.async_copy(
            table_hbm.at[idx_v.at[pl.ds(half, half)]],
            rows_v.at[pl.ds(half, half)], sem1)
        g0.wait()
        w0 = pltpu.async_copy(
            rows_v.at[pl.ds(0, half)],
            out_hbm.at[pl.ds(base, half)], semw)
        g1.wait()
        w1 = pltpu.async_copy(
            rows_v.at[pl.ds(half, half)],
            out_hbm.at[pl.ds(base + half, half)], semw)
        w0.wait()
        w1.wait()

    return _sc_gather


_sc_gather_full = _make_gather(N_TOK)


def kernel(inputs, embeddings):
    flat = jnp.reshape(inputs, (N_TOK, DIM))
    emb_t = embeddings.T
    idx = _argmin_call(flat, emb_t).reshape(N_TOK)
    table_p = jnp.pad(embeddings, ((0, 0), (0, PAD_D - DIM)))
    quantized = _sc_gather_full(table_p, idx)[:, :DIM]
    return jnp.reshape(quantized, inputs.shape)


# speedup vs baseline: 1.1414x; 1.1414x over previous
import functools

import jax
import jax.numpy as jnp
from jax import lax
from jax.experimental import pallas as pl
from jax.experimental.pallas import tpu as pltpu
from jax.experimental.pallas import tpu_sc as plsc

NUM_E = 1024
DIM = 64
N_TOK = 16 * 24 * 24
TILE = 3072
GRID = N_TOK // TILE

NC = 2
NS = 16
NW = NC * NS
B_PER_W = N_TOK // NW
N_CHUNK = 3
CHUNK = B_PER_W // N_CHUNK


def _argmin_body(x_ref, emb_t_ref, idx_ref, table_ref):
    @pl.when(pl.program_id(0) == 0)
    def _():
        et0 = emb_t_ref[...]
        table_ref[...] = jnp.concatenate(
            [et0.T, jnp.zeros((NUM_E, PAD_D - DIM), jnp.float32)], axis=1
        )

    x = x_ref[...]
    et = emb_t_ref[...]
    scores2 = lax.dot_general(
        x, et + et, (((1,), (0,)), ((), ())), preferred_element_type=jnp.float32
    )
    x2 = jnp.sum(x * x, axis=1, keepdims=True)
    e2 = jnp.sum(et * et, axis=0, keepdims=True)
    best_v = x2 + e2[:, 0:128] - scores2[:, 0:128]
    best_g = jnp.zeros((TILE, 128), jnp.float32)
    for g in range(1, NUM_E // 128):
        v = x2 + e2[:, 128 * g:128 * (g + 1)] - scores2[:, 128 * g:128 * (g + 1)]
        upd = v < best_v
        best_v = jnp.where(upd, v, best_v)
        best_g = jnp.where(upd, jnp.float32(g), best_g)
    lane = lax.broadcasted_iota(jnp.int32, (TILE, 128), 1).astype(jnp.float32)
    best_c = best_g * 128.0 + lane
    bv_t = best_v.T
    bc_t = best_c.T
    m_t = jnp.min(bv_t, axis=0, keepdims=True)
    idx_f = jnp.min(
        jnp.where(bv_t == m_t, bc_t, jnp.float32(NUM_E)), axis=0
    )
    idx_ref[...] = idx_f.astype(jnp.int32)


_argmin_call = pl.pallas_call(
    _argmin_body,
    grid=(GRID,),
    in_specs=[
        pl.BlockSpec((TILE, DIM), lambda i: (i, 0)),
        pl.BlockSpec((DIM, NUM_E), lambda i: (0, 0)),
    ],
    out_specs=[
        pl.BlockSpec((TILE,), lambda i: (i,)),
        pl.BlockSpec((1024, 128), lambda i: (0, 0)),
    ],
    out_shape=[
        jax.ShapeDtypeStruct((N_TOK,), jnp.int32),
        jax.ShapeDtypeStruct((1024, 128), jnp.float32),
    ],
)


PAD_D = 128


@functools.partial(
    pl.kernel,
    mesh=plsc.VectorSubcoreMesh(core_axis_name="c", subcore_axis_name="s"),
    out_type=jax.ShapeDtypeStruct((N_TOK, PAD_D), jnp.float32),
    scratch_types=[
        pltpu.VMEM((B_PER_W,), jnp.int32),
        pltpu.VMEM((B_PER_W, PAD_D), jnp.float32),
        pltpu.SemaphoreType.DMA,
    ],
)
def _sc_gather(table_hbm, idx_hbm, out_hbm, idx_v, rows_v, sem):
    wid = lax.axis_index("s") * NC + lax.axis_index("c")
    base = wid * B_PER_W
    pltpu.sync_copy(idx_hbm.at[pl.ds(base, B_PER_W)], idx_v)
    pltpu.async_copy(table_hbm.at[idx_v], rows_v, sem).wait()
    pltpu.sync_copy(rows_v, out_hbm.at[pl.ds(base, B_PER_W)])


def kernel(inputs, embeddings):
    flat = jnp.reshape(inputs, (N_TOK, DIM))
    emb_t = embeddings.T
    idx, table_p = _argmin_call(flat, emb_t)
    quantized = _sc_gather(table_p, idx)[:, :DIM]
    return jnp.reshape(quantized, inputs.shape)
